# baseline (speedup 1.0000x reference)
import jax
import jax.numpy as jnp
from jax import lax
from jax.experimental import pallas as pl
from jax.experimental.pallas import tpu as pltpu
from jax.experimental.pallas import tpu_sc as plsc

N = 10000
D = 128
E = 320000

NC = 2
NS = 16
NPAD = 10240
HALF = NPAD // NC
AROWS = HALF + NS
K = 80
NBUF = 4
NGRP = 3
GCHUNK = 84
NCHUNK = NGRP * GCHUNK
E_TILE = NCHUNK * K
ROWS_PER_TILE = HALF // NS
DST_PAD = 1 << 29


def _seg_body(x_hbm, src_hbm, dst_hbm, out_hbm,
              src_v, dst_v, r0, r1, r2, r3, agg, gsem, ssem):
    cid = lax.axis_index("c")
    sid = lax.axis_index("s")
    rv = [r0, r1, r2, r3]

    z = jnp.zeros((16,), jnp.float32)

    def zrow(r, carry):
        for j in range(D // 16):
            r0[r, pl.ds(j * 16, 16)] = z
        return carry

    lax.fori_loop(0, K, zrow, 0)
    row0 = sid * ROWS_PER_TILE
    for t in range(ROWS_PER_TILE // K):
        pltpu.sync_copy(r0, agg.at[pl.ds(row0 + t * K, K)])
    plsc.subcore_barrier()

    base = cid * HALF
    trash = jnp.full((16,), HALF, jnp.int32) + sid

    def gather(c, b):
        return pltpu.make_async_copy(x_hbm.at[src_v.at[c]], rv[b], gsem)

    def scatter(c, b):
        return pltpu.async_copy(rv[b], agg.at[dst_v.at[c]], ssem, add=True)

    def scatter_wait(c, b):
        pltpu.make_async_copy(rv[b], agg.at[dst_v.at[c]], ssem).wait()

    for g in range(NGRP):
        pltpu.sync_copy(src_hbm.at[sid, g], src_v)
        pltpu.sync_copy(dst_hbm.at[sid, g], dst_v)

        def rebase(r, carry):
            for j in range(K // 16):
                v = dst_v[r, pl.ds(j * 16, 16)] - base
                ok = (v >= 0) & (v < HALF)
                dst_v[r, pl.ds(j * 16, 16)] = jnp.where(ok, v, trash)
            return carry

        lax.fori_loop(0, GCHUNK, rebase, 0)

        for b in range(NBUF - 1):
            gather(b, b).start()

        def step(c, b):
            gather(c, b).wait()

            @pl.when(c >= 1)
            def _():
                scatter_wait(c - 1, (b + NBUF - 1) % NBUF)

            @pl.when(c + NBUF - 1 < GCHUNK)
            def _():
                gather(c + NBUF - 1, (b + NBUF - 1) % NBUF).start()

            scatter(c, b)

        def ring(t, carry):
            for b in range(NBUF):
                step(t * NBUF + b, b)
            return carry

        lax.fori_loop(0, GCHUNK // NBUF, ring, 0)
        scatter_wait(GCHUNK - 1, (GCHUNK - 1) % NBUF)

    plsc.subcore_barrier()

    pltpu.sync_copy(agg.at[pl.ds(row0, ROWS_PER_TILE)],
                    out_hbm.at[pl.ds(base + row0, ROWS_PER_TILE)])


@jax.jit
def _seg_sum(x, src_r, dst_r):
    mesh = plsc.VectorSubcoreMesh(core_axis_name="c", subcore_axis_name="s")
    return pl.kernel(
        _seg_body,
        out_type=jax.ShapeDtypeStruct((NPAD, D), jnp.float32),
        mesh=mesh,
        scratch_types=[
            pltpu.VMEM((GCHUNK, K), jnp.int32),
            pltpu.VMEM((GCHUNK, K), jnp.int32),
            pltpu.VMEM((K, D), jnp.float32),
            pltpu.VMEM((K, D), jnp.float32),
            pltpu.VMEM((K, D), jnp.float32),
            pltpu.VMEM((K, D), jnp.float32),
            pltpu.VMEM_SHARED((AROWS, D), jnp.float32),
            pltpu.SemaphoreType.DMA,
            pltpu.SemaphoreType.DMA,
        ],
    )(x, src_r, dst_r)


def _mlp_body(p_ref, w_ref, b_ref, o_ref):
    y = jnp.dot(p_ref[...], w_ref[...],
                preferred_element_type=jnp.float32) + b_ref[...]
    o_ref[...] = jnp.maximum(y, 0.0)


@jax.jit
def _mlp(p, W, b):
    R = 1024
    return pl.pallas_call(
        _mlp_body,
        grid=(NPAD // R,),
        in_specs=[
            pl.BlockSpec((R, D), lambda i: (i, 0)),
            pl.BlockSpec((D, D), lambda i: (0, 0)),
            pl.BlockSpec((1, D), lambda i: (0, 0)),
        ],
        out_specs=pl.BlockSpec((R, D), lambda i: (i, 0)),
        out_shape=jax.ShapeDtypeStruct((NPAD, D), jnp.float32),
    )(p, W, b.reshape(1, D))


def _pad_idx(a, fill):
    a = a.reshape(NS, E // NS)
    a = jnp.pad(a, ((0, 0), (0, E_TILE - E // NS)), constant_values=fill)
    return a.reshape(NS, NGRP, GCHUNK, K)


def kernel(h, edge_index, W1, b1, W2, b2):
    src_r = _pad_idx(edge_index[0], 0)
    dst_r = _pad_idx(edge_index[1], DST_PAD)
    a1 = _seg_sum(h, src_r, dst_r)
    x = _mlp(a1, W1, b1)
    a2 = _seg_sum(x, src_r, dst_r)
    return _mlp(a2, W2, b2)[:N]


# speedup vs baseline: 1.9009x; 1.9009x over previous
import jax
import jax.numpy as jnp
from jax import lax
from jax.experimental import pallas as pl
from jax.experimental.pallas import tpu as pltpu
from jax.experimental.pallas import tpu_sc as plsc

N = 10000
D = 128
E = 320000

NC = 2
NS = 16
DH = D // NC
NPAD = 10240
AROWS = NPAD + NS
K = 80
NBUF = 4
NGRP = 3
GCHUNK = 84
NCHUNK = NGRP * GCHUNK
E_TILE = NCHUNK * K
ROWS_PER_TILE = NPAD // NS


def _seg_body(x_hbm, src_hbm, dst_hbm, out_hbm,
              src_v, dst_v, r0, r1, r2, r3, agg, gsem, ssem):
    cid = lax.axis_index("c")
    sid = lax.axis_index("s")
    rv = [r0, r1, r2, r3]

    z = jnp.zeros((16,), jnp.float32)

    def zrow(r, carry):
        for j in range(DH // 16):
            r0[r, pl.ds(j * 16, 16)] = z
        return carry

    lax.fori_loop(0, K, zrow, 0)
    row0 = sid * ROWS_PER_TILE
    for t in range(ROWS_PER_TILE // K):
        pltpu.sync_copy(r0, agg.at[pl.ds(row0 + t * K, K)])
    plsc.subcore_barrier()

    def gather(c, b):
        return pltpu.make_async_copy(x_hbm.at[src_v.at[c]], rv[b], gsem)

    def scatter(c, b):
        return pltpu.async_copy(rv[b], agg.at[dst_v.at[c]], ssem, add=True)

    def scatter_wait(c, b):
        pltpu.make_async_copy(rv[b], agg.at[dst_v.at[c]], ssem).wait()

    for g in range(NGRP):
        pltpu.sync_copy(src_hbm.at[cid, sid, g], src_v)
        pltpu.sync_copy(dst_hbm.at[sid, g], dst_v)

        for b in range(NBUF - 1):
            gather(b, b).start()

        def step(c, b):
            gather(c, b).wait()

            @pl.when(c >= 1)
            def _():
                scatter_wait(c - 1, (b + NBUF - 1) % NBUF)

            @pl.when(c + NBUF - 1 < GCHUNK)
            def _():
                gather(c + NBUF - 1, (b + NBUF - 1) % NBUF).start()

            scatter(c, b)

        def ring(t, carry):
            for b in range(NBUF):
                step(t * NBUF + b, b)
            return carry

        lax.fori_loop(0, GCHUNK // NBUF, ring, 0)
        scatter_wait(GCHUNK - 1, (GCHUNK - 1) % NBUF)

    plsc.subcore_barrier()

    pltpu.sync_copy(agg.at[pl.ds(row0, ROWS_PER_TILE)],
                    out_hbm.at[pl.ds(cid * NPAD + row0, ROWS_PER_TILE)])


@jax.jit
def _seg_sum(xs, src_r, dst_r):
    mesh = plsc.VectorSubcoreMesh(core_axis_name="c", subcore_axis_name="s")
    return pl.kernel(
        _seg_body,
        out_type=jax.ShapeDtypeStruct((NC * NPAD, DH), jnp.float32),
        mesh=mesh,
        compiler_params=pltpu.CompilerParams(use_tc_tiling_on_sc=False),
        scratch_types=[
            pltpu.VMEM((GCHUNK, K), jnp.int32),
            pltpu.VMEM((GCHUNK, K), jnp.int32),
            pltpu.VMEM((K, DH), jnp.float32),
            pltpu.VMEM((K, DH), jnp.float32),
            pltpu.VMEM((K, DH), jnp.float32),
            pltpu.VMEM((K, DH), jnp.float32),
            pltpu.VMEM_SHARED((AROWS, DH), jnp.float32),
            pltpu.SemaphoreType.DMA,
            pltpu.SemaphoreType.DMA,
        ],
    )(xs, src_r, dst_r)


def _mlp_mid_body(p_ref, w_ref, b_ref, o_ref):
    y = (jnp.dot(p_ref[0], w_ref[pl.ds(0, DH)],
                 preferred_element_type=jnp.float32)
         + jnp.dot(p_ref[1], w_ref[pl.ds(DH, DH)],
                   preferred_element_type=jnp.float32)
         + b_ref[...])
    y = jnp.maximum(y, 0.0)
    o_ref[0] = y[:, :DH]
    o_ref[1] = y[:, DH:]


@jax.jit
def _mlp_mid(p, W, b):
    R = 1024
    return pl.pallas_call(
        _mlp_mid_body,
        grid=(NPAD // R,),
        in_specs=[
            pl.BlockSpec((2, R, DH), lambda i: (0, i, 0)),
            pl.BlockSpec((D, D), lambda i: (0, 0)),
            pl.BlockSpec((1, D), lambda i: (0, 0)),
        ],
        out_specs=pl.BlockSpec((2, R, DH), lambda i: (0, i, 0)),
        out_shape=jax.ShapeDtypeStruct((2, NPAD, DH), jnp.float32),
    )(p, W, b.reshape(1, D))


def _mlp_fin_body(p_ref, w_ref, b_ref, o_ref):
    y = (jnp.dot(p_ref[0], w_ref[pl.ds(0, DH)],
                 preferred_element_type=jnp.float32)
         + jnp.dot(p_ref[1], w_ref[pl.ds(DH, DH)],
                   preferred_element_type=jnp.float32)
         + b_ref[...])
    o_ref[...] = jnp.maximum(y, 0.0)


@jax.jit
def _mlp_fin(p, W, b):
    R = 1024
    return pl.pallas_call(
        _mlp_fin_body,
        grid=(NPAD // R,),
        in_specs=[
            pl.BlockSpec((2, R, DH), lambda i: (0, i, 0)),
            pl.BlockSpec((D, D), lambda i: (0, 0)),
            pl.BlockSpec((1, D), lambda i: (0, 0)),
        ],
        out_specs=pl.BlockSpec((R, D), lambda i: (i, 0)),
        out_shape=jax.ShapeDtypeStruct((NPAD, D), jnp.float32),
    )(p, W, b.reshape(1, D))


def kernel(h, edge_index, W1, b1, W2, b2):
    src = edge_index[0].reshape(NS, E // NS)
    src = jnp.pad(src, ((0, 0), (0, E_TILE - E // NS)))
    src = src.reshape(1, NS, NGRP, GCHUNK, K)
    src_r = jnp.concatenate([src, src + NPAD], axis=0)

    dst = edge_index[1].reshape(NS, E // NS)
    trash = jnp.broadcast_to(
        (NPAD + jnp.arange(NS, dtype=jnp.int32))[:, None],
        (NS, E_TILE - E // NS))
    dst_r = jnp.concatenate([dst, trash], axis=1)
    dst_r = dst_r.reshape(NS, NGRP, GCHUNK, K)

    hs = jnp.concatenate(
        [jnp.pad(h[:, :DH], ((0, NPAD - N), (0, 0))),
         jnp.pad(h[:, DH:], ((0, NPAD - N), (0, 0)))], axis=0)

    a1 = _seg_sum(hs, src_r, dst_r).reshape(NC, NPAD, DH)
    x = _mlp_mid(a1, W1, b1)
    a2 = _seg_sum(x.reshape(NC * NPAD, DH), src_r, dst_r)
    return _mlp_fin(a2.reshape(NC, NPAD, DH), W2, b2)[:N]
